# Initial kernel scaffold; baseline (speedup 1.0000x reference)
import jax
import jax.numpy as jnp
from jax.experimental import pallas as pl


def kernel(x, params):
    raise NotImplementedError("write your pallas kernel here")


# speedup vs baseline: 1.3266x; 1.3266x over previous
import functools
import math

import numpy as np
import jax
import jax.numpy as jnp
from jax.experimental import pallas as pl
from jax.experimental.pallas import tpu as pltpu

N_HEADS = 16
SAMPLE_FACTOR = 5
PREC = jax.lax.Precision.HIGHEST

TM = 512
TN = 1024
TK = 1024
QBLK = 256


def _posenc_np(Lx, d_model):
    position = np.arange(Lx, dtype=np.float64)[:, None]
    div_term = np.exp(np.arange(0, d_model, 2, dtype=np.float64)
                      * -(np.log(10000.0) / d_model))
    pe = np.zeros((Lx, d_model), dtype=np.float64)
    pe[:, 0::2] = np.sin(position * div_term)
    pe[:, 1::2] = np.cos(position * div_term)
    return pe.astype(np.float32)


@functools.lru_cache(maxsize=None)
def _sample_counts(L, U, layer):
    with jax.ensure_compile_time_eval():
        key = jax.random.fold_in(jax.random.key(42), layer)
        idx = np.asarray(jax.random.randint(key, (L, U), 0, L))
    cnt = np.zeros((L, L), np.int8)
    np.add.at(cnt, (np.arange(L)[:, None], idx), 1)
    return cnt


def _dot(a, b):
    return jax.lax.dot_general(a, b, (((1,), (1,)), ((), ())),
                               preferred_element_type=jnp.float32,
                               precision=PREC)


def _mm_bias_kernel(a_ref, w_ref, b_ref, o_ref, acc_ref, *, nk, act):
    k = pl.program_id(2)

    @pl.when(k == 0)
    def _():
        acc_ref[...] = jnp.zeros_like(acc_ref)

    acc_ref[...] += _dot(a_ref[...], w_ref[...])

    @pl.when(k == nk - 1)
    def _():
        r = acc_ref[...] + b_ref[...]
        if act == "gelu":
            r = r * 0.5 * (1.0 + jax.lax.erf(r * (1.0 / math.sqrt(2.0))))
        o_ref[...] = r


def _mm_bias(a, w, b, act=None):
    M, K = a.shape
    N = w.shape[0]
    nk = K // TK
    return pl.pallas_call(
        functools.partial(_mm_bias_kernel, nk=nk, act=act),
        grid=(M // TM, N // TN, nk),
        in_specs=[
            pl.BlockSpec((TM, TK), lambda m, n, k: (m, k)),
            pl.BlockSpec((TN, TK), lambda m, n, k: (n, k)),
            pl.BlockSpec((1, TN), lambda m, n, k: (0, n)),
        ],
        out_specs=pl.BlockSpec((TM, TN), lambda m, n, k: (m, n)),
        out_shape=jax.ShapeDtypeStruct((M, N), jnp.float32),
        scratch_shapes=[pltpu.VMEM((TM, TN), jnp.float32)],
        compiler_params=pltpu.CompilerParams(
            dimension_semantics=("parallel", "arbitrary", "arbitrary")),
    )(a, w, b)


def _mm_add_kernel(a_ref, w_ref, add_ref, o_ref):
    o_ref[...] = _dot(a_ref[...], w_ref[...]) + add_ref[...]


def _mm_add(a, w, add_rows, L):
    M, K = a.shape
    N = w.shape[0]
    nlt = L // TM
    return pl.pallas_call(
        _mm_add_kernel,
        grid=(M // TM,),
        in_specs=[
            pl.BlockSpec((TM, K), lambda m: (m, 0)),
            pl.BlockSpec((N, K), lambda m: (0, 0)),
            pl.BlockSpec((TM, N), lambda m: (m % nlt, 0)),
        ],
        out_specs=pl.BlockSpec((TM, N), lambda m: (m, 0)),
        out_shape=jax.ShapeDtypeStruct((M, N), jnp.float32),
    )(a, w, add_rows)


def _ln(r, g, b):
    mu = jnp.mean(r, axis=-1, keepdims=True)
    d = r - mu
    var = jnp.mean(d * d, axis=-1, keepdims=True)
    return d * jax.lax.rsqrt(var + 1e-5) * g + b


def _mm_res_ln_kernel(a_ref, w_ref, b_ref, res_ref, g_ref, be_ref,
                      o_ref, acc_ref, *, nk):
    k = pl.program_id(1)

    @pl.when(k == 0)
    def _():
        acc_ref[...] = jnp.zeros_like(acc_ref)

    acc_ref[...] += _dot(a_ref[...], w_ref[...])

    @pl.when(k == nk - 1)
    def _():
        r = acc_ref[...] + b_ref[...] + res_ref[...]
        o_ref[...] = _ln(r, g_ref[...], be_ref[...])


def _mm_res_ln(a, w, b, res, g, be):
    M, K = a.shape
    N = w.shape[0]
    nk = K // TK
    return pl.pallas_call(
        functools.partial(_mm_res_ln_kernel, nk=nk),
        grid=(M // TM, nk),
        in_specs=[
            pl.BlockSpec((TM, TK), lambda m, k: (m, k)),
            pl.BlockSpec((N, TK), lambda m, k: (0, k)),
            pl.BlockSpec((1, N), lambda m, k: (0, 0)),
            pl.BlockSpec((TM, N), lambda m, k: (m, 0)),
            pl.BlockSpec((1, N), lambda m, k: (0, 0)),
            pl.BlockSpec((1, N), lambda m, k: (0, 0)),
        ],
        out_specs=pl.BlockSpec((TM, N), lambda m, k: (m, 0)),
        out_shape=jax.ShapeDtypeStruct((M, N), jnp.float32),
        scratch_shapes=[pltpu.VMEM((TM, N), jnp.float32)],
        compiler_params=pltpu.CompilerParams(
            dimension_semantics=("parallel", "arbitrary")),
    )(a, w, b, res, g, be)


def _ln_mm_kernel(a_ref, g_ref, be_ref, w_ref, b_ref, o_ref):
    xn = _ln(a_ref[...], g_ref[...], be_ref[...])
    o_ref[...] = _dot(xn, w_ref[...]) + b_ref[...]


def _ln_mm(a, g, be, w, b):
    M, K = a.shape
    N = w.shape[0]
    return pl.pallas_call(
        _ln_mm_kernel,
        grid=(M // TM,),
        in_specs=[
            pl.BlockSpec((TM, K), lambda m: (m, 0)),
            pl.BlockSpec((1, K), lambda m: (0, 0)),
            pl.BlockSpec((1, K), lambda m: (0, 0)),
            pl.BlockSpec((N, K), lambda m: (0, 0)),
            pl.BlockSpec((1, N), lambda m: (0, 0)),
        ],
        out_specs=pl.BlockSpec((TM, N), lambda m: (m, 0)),
        out_shape=jax.ShapeDtypeStruct((M, N), jnp.float32),
    )(a, g, be, w, b)


def _attn_m_kernel(q_ref, k_ref, cnt_ref, m_ref, *, L):
    kk = k_ref[0]
    for i in range(L // QBLK):
        qb = q_ref[0, i * QBLK:(i + 1) * QBLK, :]
        s = _dot(qb, kk)
        cf = cnt_ref[i * QBLK:(i + 1) * QBLK, :].astype(jnp.float32)
        mx = jnp.max(jnp.where(cf > 0, s, -jnp.inf), axis=1)
        sm = jnp.sum(s * cf, axis=1)
        m_ref[0, 0, i * QBLK:(i + 1) * QBLK] = mx - sm * (1.0 / L)


def _attn_m(q, k, cnt):
    BH, L, D = q.shape
    return pl.pallas_call(
        functools.partial(_attn_m_kernel, L=L),
        grid=(BH,),
        in_specs=[
            pl.BlockSpec((1, L, D), lambda b: (b, 0, 0)),
            pl.BlockSpec((1, L, D), lambda b: (b, 0, 0)),
            pl.BlockSpec((L, L), lambda b: (0, 0)),
        ],
        out_specs=pl.BlockSpec((1, 1, L), lambda b: (b, 0, 0)),
        out_shape=jax.ShapeDtypeStruct((BH, 1, L), jnp.float32),
    )(q, k, cnt)


def _topk_kernel(m_ref, p_ref, *, u, up, L):
    p_ref[...] = jnp.zeros_like(p_ref)
    m0 = m_ref[:, 0, :]
    iota = jax.lax.broadcasted_iota(jnp.int32, m0.shape, 1)

    def body(i, mcur):
        mx = jnp.max(mcur, axis=1, keepdims=True)
        sel = mcur == mx
        idx = jnp.min(jnp.where(sel, iota, L), axis=1, keepdims=True)
        onehot = iota == idx
        p_ref[:, pl.ds(i, 1), :] = onehot.astype(jnp.float32)[:, None, :]
        return jnp.where(onehot, -jnp.inf, mcur)

    jax.lax.fori_loop(0, u, body, m0)


def _topk(m, u, up):
    BH, _, L = m.shape
    return pl.pallas_call(
        functools.partial(_topk_kernel, u=u, up=up, L=L),
        grid=(1,),
        in_specs=[pl.BlockSpec((BH, 1, L), lambda i: (0, 0, 0))],
        out_specs=pl.BlockSpec((BH, up, L), lambda i: (0, 0, 0)),
        out_shape=jax.ShapeDtypeStruct((BH, up, L), jnp.float32),
    )(m)


def _attn_ctx_kernel(p_ref, q_ref, k_ref, v_ref, o_ref, *, D):
    p = p_ref[0]
    q = q_ref[0]
    kk = k_ref[0]
    v = v_ref[0]
    qr = jnp.dot(p, q, preferred_element_type=jnp.float32,
                 precision=PREC)

    s = _dot(qr, kk) * (1.0 / math.sqrt(D))
    s = s - jnp.max(s, axis=1, keepdims=True)
    e = jnp.exp(s)
    a = e / jnp.sum(e, axis=1, keepdims=True)
    upd = jnp.dot(a, v, preferred_element_type=jnp.float32, precision=PREC)
    vm = jnp.mean(v, axis=0, keepdims=True)
    delta = upd - vm
    corr = jax.lax.dot_general(p, delta, (((0,), (0,)), ((), ())),
                               preferred_element_type=jnp.float32,
                               precision=PREC)
    o_ref[0] = vm + corr


def _attn_ctx(p, q, k, v):
    BH, L, D = q.shape
    up = p.shape[1]
    return pl.pallas_call(
        functools.partial(_attn_ctx_kernel, D=D),
        grid=(BH,),
        in_specs=[
            pl.BlockSpec((1, up, L), lambda b: (b, 0, 0)),
            pl.BlockSpec((1, L, D), lambda b: (b, 0, 0)),
            pl.BlockSpec((1, L, D), lambda b: (b, 0, 0)),
            pl.BlockSpec((1, L, D), lambda b: (b, 0, 0)),
        ],
        out_specs=pl.BlockSpec((1, L, D), lambda b: (b, 0, 0)),
        out_shape=jax.ShapeDtypeStruct((BH, L, D), jnp.float32),
    )(p, q, k, v)


def _layer(h, p, layer_idx, B, L, dm):
    H = N_HEADS
    D = dm // H
    M = B * L
    wqkv = jnp.concatenate([p['Wq'], p['Wk'], p['Wv']], axis=0)
    bqkv = jnp.concatenate([p['bq'], p['bk'], p['bv']])[None, :]
    qkv = _mm_bias(h, wqkv, bqkv)

    def heads(zcol):
        z = qkv[:, zcol * dm:(zcol + 1) * dm]
        return z.reshape(B, L, H, D).transpose(0, 2, 1, 3).reshape(B * H, L, D)

    q, k, v = heads(0), heads(1), heads(2)

    u = min(SAMPLE_FACTOR * int(math.ceil(math.log(L))), L)
    up_pad = (u + 7) // 8 * 8
    cnt = jnp.asarray(_sample_counts(L, u, layer_idx))
    m = _attn_m(q, k, cnt)
    psel = _topk(m, u, up_pad)
    ctx = _attn_ctx(psel, q, k, v)

    ctx2 = ctx.reshape(B, H, L, D).reshape(B, L, H * D).reshape(M, dm)
    h1 = _mm_res_ln(ctx2, p['Wo'], p['bo'][None, :], h,
                    p['g1'][None, :], p['be1'][None, :])
    y = _mm_bias(h1, p['W1'], p['bf1'][None, :], act="gelu")
    h2 = _mm_res_ln(y, p['W2'], p['bf2'][None, :], h1,
                    p['g2'][None, :], p['be2'][None, :])
    return h2


def kernel(x, params):
    B, L, Cin = x.shape
    dm = params['Wp'].shape[1]
    M = B * L

    xp3 = jnp.concatenate(
        [jnp.roll(x, 1, axis=1), x, jnp.roll(x, -1, axis=1)], axis=-1)
    w3 = jnp.concatenate([params['Wemb'][:, :, 0], params['Wemb'][:, :, 1],
                          params['Wemb'][:, :, 2]], axis=1)
    pe = jnp.asarray(_posenc_np(L, dm))
    h = _mm_add(xp3.reshape(M, 3 * Cin), w3, pe, L)

    for i, p in enumerate(params['layers']):
        h = _layer(h, p, i, B, L, dm)

    out = _ln_mm(h, params['gN'][None, :], params['bN'][None, :],
                 params['Wp'], params['bp'][None, :])
    return out.reshape(B, L, -1)


# speedup vs baseline: 2.0665x; 1.5578x over previous
import functools
import math

import numpy as np
import jax
import jax.numpy as jnp
from jax.experimental import pallas as pl
from jax.experimental.pallas import tpu as pltpu

N_HEADS = 16
SAMPLE_FACTOR = 5
PREC = jax.lax.Precision.HIGHEST

TM = 512
TN = 1024
TK = 1024
QBLK = 256


def _posenc_np(Lx, d_model):
    position = np.arange(Lx, dtype=np.float64)[:, None]
    div_term = np.exp(np.arange(0, d_model, 2, dtype=np.float64)
                      * -(np.log(10000.0) / d_model))
    pe = np.zeros((Lx, d_model), dtype=np.float64)
    pe[:, 0::2] = np.sin(position * div_term)
    pe[:, 1::2] = np.cos(position * div_term)
    return pe.astype(np.float32)


@functools.lru_cache(maxsize=None)
def _sample_counts(L, U, layer):
    with jax.ensure_compile_time_eval():
        try:
            dev_ctx = jax.default_device(jax.devices("cpu")[0])
        except Exception:
            import contextlib
            dev_ctx = contextlib.nullcontext()
        with dev_ctx:
            key = jax.random.fold_in(jax.random.key(42), layer)
            idx = np.asarray(jax.random.randint(key, (L, U), 0, L))
    cnt = np.zeros((L, L), np.int8)
    np.add.at(cnt, (np.arange(L)[:, None], idx), 1)
    return cnt


def _split(a):
    hi = a.astype(jnp.bfloat16)
    lo = (a - hi.astype(jnp.float32)).astype(jnp.bfloat16)
    return hi, lo


def _dot3_g(a, b, dims):
    a_hi, a_lo = _split(a)
    b_hi, b_lo = _split(b)

    def d(x, y):
        return jax.lax.dot_general(x, y, (dims, ((), ())),
                                   preferred_element_type=jnp.float32)

    return d(a_hi, b_hi) + d(a_hi, b_lo) + d(a_lo, b_hi)


def _dot(a, b):
    return _dot3_g(a, b, ((1,), (1,)))


def _mm_bias_kernel(a_ref, w_ref, b_ref, o_ref, acc_ref, *, nk, act):
    k = pl.program_id(2)

    @pl.when(k == 0)
    def _():
        acc_ref[...] = jnp.zeros_like(acc_ref)

    acc_ref[...] += _dot(a_ref[...], w_ref[...])

    @pl.when(k == nk - 1)
    def _():
        r = acc_ref[...] + b_ref[...]
        if act == "gelu":
            r = r * 0.5 * (1.0 + jax.lax.erf(r * (1.0 / math.sqrt(2.0))))
        o_ref[...] = r


def _mm_bias(a, w, b, act=None):
    M, K = a.shape
    N = w.shape[0]
    nk = K // TK
    return pl.pallas_call(
        functools.partial(_mm_bias_kernel, nk=nk, act=act),
        grid=(M // TM, N // TN, nk),
        in_specs=[
            pl.BlockSpec((TM, TK), lambda m, n, k: (m, k)),
            pl.BlockSpec((TN, TK), lambda m, n, k: (n, k)),
            pl.BlockSpec((1, TN), lambda m, n, k: (0, n)),
        ],
        out_specs=pl.BlockSpec((TM, TN), lambda m, n, k: (m, n)),
        out_shape=jax.ShapeDtypeStruct((M, N), jnp.float32),
        scratch_shapes=[pltpu.VMEM((TM, TN), jnp.float32)],
        compiler_params=pltpu.CompilerParams(
            dimension_semantics=("parallel", "arbitrary", "arbitrary")),
    )(a, w, b)


def _mm_add_kernel(a_ref, w_ref, add_ref, o_ref):
    o_ref[...] = _dot(a_ref[...], w_ref[...]) + add_ref[...]


def _mm_add(a, w, add_rows, L):
    M, K = a.shape
    N = w.shape[0]
    nlt = L // TM
    return pl.pallas_call(
        _mm_add_kernel,
        grid=(M // TM,),
        in_specs=[
            pl.BlockSpec((TM, K), lambda m: (m, 0)),
            pl.BlockSpec((N, K), lambda m: (0, 0)),
            pl.BlockSpec((TM, N), lambda m: (m % nlt, 0)),
        ],
        out_specs=pl.BlockSpec((TM, N), lambda m: (m, 0)),
        out_shape=jax.ShapeDtypeStruct((M, N), jnp.float32),
    )(a, w, add_rows)


def _ln(r, g, b):
    mu = jnp.mean(r, axis=-1, keepdims=True)
    d = r - mu
    var = jnp.mean(d * d, axis=-1, keepdims=True)
    return d * jax.lax.rsqrt(var + 1e-5) * g + b


def _mm_res_ln_kernel(a_ref, w_ref, b_ref, res_ref, g_ref, be_ref,
                      o_ref, acc_ref, *, nk):
    k = pl.program_id(1)

    @pl.when(k == 0)
    def _():
        acc_ref[...] = jnp.zeros_like(acc_ref)

    acc_ref[...] += _dot(a_ref[...], w_ref[...])

    @pl.when(k == nk - 1)
    def _():
        r = acc_ref[...] + b_ref[...] + res_ref[...]
        o_ref[...] = _ln(r, g_ref[...], be_ref[...])


def _mm_res_ln(a, w, b, res, g, be):
    M, K = a.shape
    N = w.shape[0]
    nk = K // TK
    return pl.pallas_call(
        functools.partial(_mm_res_ln_kernel, nk=nk),
        grid=(M // TM, nk),
        in_specs=[
            pl.BlockSpec((TM, TK), lambda m, k: (m, k)),
            pl.BlockSpec((N, TK), lambda m, k: (0, k)),
            pl.BlockSpec((1, N), lambda m, k: (0, 0)),
            pl.BlockSpec((TM, N), lambda m, k: (m, 0)),
            pl.BlockSpec((1, N), lambda m, k: (0, 0)),
            pl.BlockSpec((1, N), lambda m, k: (0, 0)),
        ],
        out_specs=pl.BlockSpec((TM, N), lambda m, k: (m, 0)),
        out_shape=jax.ShapeDtypeStruct((M, N), jnp.float32),
        scratch_shapes=[pltpu.VMEM((TM, N), jnp.float32)],
        compiler_params=pltpu.CompilerParams(
            dimension_semantics=("parallel", "arbitrary")),
    )(a, w, b, res, g, be)


def _ln_mm_kernel(a_ref, g_ref, be_ref, w_ref, b_ref, o_ref):
    xn = _ln(a_ref[...], g_ref[...], be_ref[...])
    o_ref[...] = _dot(xn, w_ref[...]) + b_ref[...]


def _ln_mm(a, g, be, w, b):
    M, K = a.shape
    N = w.shape[0]
    return pl.pallas_call(
        _ln_mm_kernel,
        grid=(M // TM,),
        in_specs=[
            pl.BlockSpec((TM, K), lambda m: (m, 0)),
            pl.BlockSpec((1, K), lambda m: (0, 0)),
            pl.BlockSpec((1, K), lambda m: (0, 0)),
            pl.BlockSpec((N, K), lambda m: (0, 0)),
            pl.BlockSpec((1, N), lambda m: (0, 0)),
        ],
        out_specs=pl.BlockSpec((TM, N), lambda m: (m, 0)),
        out_shape=jax.ShapeDtypeStruct((M, N), jnp.float32),
    )(a, g, be, w, b)


def _attn_m_kernel(q_ref, k_ref, cnt_ref, m_ref, *, L):
    kk = k_ref[0]
    for i in range(L // QBLK):
        qb = q_ref[0, i * QBLK:(i + 1) * QBLK, :]
        s = _dot(qb, kk)
        cf = cnt_ref[i * QBLK:(i + 1) * QBLK, :].astype(jnp.float32)
        mx = jnp.max(jnp.where(cf > 0, s, -jnp.inf), axis=1)
        sm = jnp.sum(s * cf, axis=1)
        m_ref[0, 0, i * QBLK:(i + 1) * QBLK] = mx - sm * (1.0 / L)


def _attn_m(q, k, cnt):
    BH, L, D = q.shape
    return pl.pallas_call(
        functools.partial(_attn_m_kernel, L=L),
        grid=(BH,),
        in_specs=[
            pl.BlockSpec((1, L, D), lambda b: (b, 0, 0)),
            pl.BlockSpec((1, L, D), lambda b: (b, 0, 0)),
            pl.BlockSpec((L, L), lambda b: (0, 0)),
        ],
        out_specs=pl.BlockSpec((1, 1, L), lambda b: (b, 0, 0)),
        out_shape=jax.ShapeDtypeStruct((BH, 1, L), jnp.float32),
    )(q, k, cnt)


def _topk_kernel(m_ref, p_ref, *, u, up, L):
    p_ref[...] = jnp.zeros_like(p_ref)
    m0 = m_ref[:, 0, :]
    iota = jax.lax.broadcasted_iota(jnp.int32, m0.shape, 1)

    def body(i, mcur):
        mx = jnp.max(mcur, axis=1, keepdims=True)
        sel = mcur == mx
        idx = jnp.min(jnp.where(sel, iota, L), axis=1, keepdims=True)
        onehot = iota == idx
        p_ref[:, pl.ds(i, 1), :] = onehot.astype(jnp.float32)[:, None, :]
        return jnp.where(onehot, -jnp.inf, mcur)

    jax.lax.fori_loop(0, u, body, m0)


def _topk(m, u, up):
    BH, _, L = m.shape
    return pl.pallas_call(
        functools.partial(_topk_kernel, u=u, up=up, L=L),
        grid=(1,),
        in_specs=[pl.BlockSpec((BH, 1, L), lambda i: (0, 0, 0))],
        out_specs=pl.BlockSpec((BH, up, L), lambda i: (0, 0, 0)),
        out_shape=jax.ShapeDtypeStruct((BH, up, L), jnp.float32),
    )(m)


def _attn_ctx_kernel(p_ref, q_ref, k_ref, v_ref, o_ref, *, D):
    p = p_ref[0]
    q = q_ref[0]
    kk = k_ref[0]
    v = v_ref[0]
    qr = _dot3_g(p, q, ((1,), (0,)))
    s = _dot(qr, kk) * (1.0 / math.sqrt(D))
    s = s - jnp.max(s, axis=1, keepdims=True)
    e = jnp.exp(s)
    a = e / jnp.sum(e, axis=1, keepdims=True)
    upd = _dot3_g(a, v, ((1,), (0,)))
    vm = jnp.mean(v, axis=0, keepdims=True)
    delta = upd - vm
    corr = _dot3_g(p, delta, ((0,), (0,)))
    o_ref[0] = vm + corr


def _attn_ctx(p, q, k, v):
    BH, L, D = q.shape
    up = p.shape[1]
    return pl.pallas_call(
        functools.partial(_attn_ctx_kernel, D=D),
        grid=(BH,),
        in_specs=[
            pl.BlockSpec((1, up, L), lambda b: (b, 0, 0)),
            pl.BlockSpec((1, L, D), lambda b: (b, 0, 0)),
            pl.BlockSpec((1, L, D), lambda b: (b, 0, 0)),
            pl.BlockSpec((1, L, D), lambda b: (b, 0, 0)),
        ],
        out_specs=pl.BlockSpec((1, L, D), lambda b: (b, 0, 0)),
        out_shape=jax.ShapeDtypeStruct((BH, L, D), jnp.float32),
    )(p, q, k, v)


def _layer(h, p, layer_idx, B, L, dm):
    H = N_HEADS
    D = dm // H
    M = B * L
    wqkv = jnp.concatenate([p['Wq'], p['Wk'], p['Wv']], axis=0)
    bqkv = jnp.concatenate([p['bq'], p['bk'], p['bv']])[None, :]
    qkv = _mm_bias(h, wqkv, bqkv)

    def heads(zcol):
        z = qkv[:, zcol * dm:(zcol + 1) * dm]
        return z.reshape(B, L, H, D).transpose(0, 2, 1, 3).reshape(B * H, L, D)

    q, k, v = heads(0), heads(1), heads(2)

    u = min(SAMPLE_FACTOR * int(math.ceil(math.log(L))), L)
    up_pad = (u + 7) // 8 * 8
    cnt = jnp.asarray(_sample_counts(L, u, layer_idx))
    m = _attn_m(q, k, cnt)
    psel = _topk(m, u, up_pad)
    ctx = _attn_ctx(psel, q, k, v)

    ctx2 = ctx.reshape(B, H, L, D).reshape(B, L, H * D).reshape(M, dm)
    h1 = _mm_res_ln(ctx2, p['Wo'], p['bo'][None, :], h,
                    p['g1'][None, :], p['be1'][None, :])
    y = _mm_bias(h1, p['W1'], p['bf1'][None, :], act="gelu")
    h2 = _mm_res_ln(y, p['W2'], p['bf2'][None, :], h1,
                    p['g2'][None, :], p['be2'][None, :])
    return h2


def kernel(x, params):
    B, L, Cin = x.shape
    dm = params['Wp'].shape[1]
    M = B * L

    xp3 = jnp.concatenate(
        [jnp.roll(x, 1, axis=1), x, jnp.roll(x, -1, axis=1)], axis=-1)
    w3 = jnp.concatenate([params['Wemb'][:, :, 0], params['Wemb'][:, :, 1],
                          params['Wemb'][:, :, 2]], axis=1)
    pe = jnp.asarray(_posenc_np(L, dm))
    h = _mm_add(xp3.reshape(M, 3 * Cin), w3, pe, L)

    for i, p in enumerate(params['layers']):
        h = _layer(h, p, i, B, L, dm)

    out = _ln_mm(h, params['gN'][None, :], params['bN'][None, :],
                 params['Wp'], params['bp'][None, :])
    return out.reshape(B, L, -1)
